# baseline (speedup 1.0000x reference)
import functools

import jax
import jax.numpy as jnp
from jax import lax
from jax.experimental import pallas as pl
from jax.experimental.pallas import tpu as pltpu
from jax.experimental.pallas import tpu_sc as plsc

N_NODES_C = 10000
D_FEAT_C = 128
N_SEG = 64
SEG_PER_W = 2
CHUNK = 40
PAD_N = 10240


def _sc_body(x_hbm, batch_hbm, mean_hbm, batch_v, buf_a, buf_b, buf_c, buf_d,
             mean_v, sem_a, sem_b, sem_c, sem_d):
    wid = lax.axis_index("s") * 2 + lax.axis_index("c")

    pltpu.sync_copy(batch_hbm, batch_v.at[pl.ds(0, N_NODES_C)])
    pad_vec = jnp.full((16,), 127, dtype=jnp.int32)
    for i in range((PAD_N - N_NODES_C) // 16):
        batch_v[pl.ds(N_NODES_C + i * 16, 16)] = pad_vec

    def count_lt(t):
        def step(_, lohi):
            lo, hi = lohi
            mid = lax.div(lo + hi, 2)
            last = batch_v[pl.ds(mid * 16, 16)][15]
            found = last >= t
            return jnp.where(found, lo, mid + 1), jnp.where(found, mid, hi)

        lo, _ = lax.fori_loop(0, 10, step, (jnp.int32(0), jnp.int32(PAD_N // 16)))
        v = batch_v[pl.ds(lo * 16, 16)]
        return lo * 16 + jnp.sum(jnp.where(v < t, 1, 0))

    t0 = (wid * 2).astype(jnp.int32)
    a_row = count_lt(t0)
    m_row = count_lt(t0 + 1)
    e_row = count_lt(t0 + 2)

    c_start = lax.div(a_row, CHUNK)
    c_end = lax.div(e_row + (CHUNK - 1), CHUNK)
    nc = c_end - c_start

    def issue(c, buf, sem):
        base = jnp.minimum(c * CHUNK, N_NODES_C - CHUNK)
        pltpu.async_copy(x_hbm.at[pl.ds(base, CHUNK)], buf, sem)

    def drain(buf, sem):
        pltpu.make_async_copy(x_hbm.at[pl.ds(0, CHUNK)], buf, sem).wait()

    def process(buf, c, acc):
        base = jnp.minimum(c * CHUNK, N_NODES_C - CHUNK)
        row_lo = jnp.maximum(a_row, c * CHUNK)
        row_hi = jnp.minimum(e_row, (c + 1) * CHUNK)
        mid = jnp.minimum(jnp.maximum(m_row, row_lo), row_hi)

        def row(i, a):
            return tuple(a[k] + buf[i, pl.ds(k * 16, 16)] for k in range(8))

        def seg_sum(lo, hi, a):
            n4 = jnp.maximum(lax.div(hi - lo, 4), 0)

            def quad(g, a):
                i = lo + 4 * g
                for r in range(4):
                    a = row(i + r, a)
                return a

            a = lax.fori_loop(0, n4, quad, a)
            return lax.fori_loop(lo + 4 * n4, hi, row, a)

        acc0 = seg_sum(row_lo - base, mid - base, tuple(acc[:8]))
        acc1 = seg_sum(mid - base, row_hi - base, tuple(acc[8:]))
        return acc0 + acc1

    slots = ((buf_a, sem_a), (buf_b, sem_b), (buf_c, sem_c), (buf_d, sem_d))

    @pl.when(nc > 0)
    def _():
        issue(c_start, buf_a, sem_a)
        issue(c_start + 1, buf_b, sem_b)
        issue(c_start + 2, buf_c, sem_c)

    def quad_body(g, acc):
        c0g = c_start + 4 * g
        for b in range(4):
            buf, sem = slots[b]
            nbuf, nsem = slots[(b + 3) % 4]
            drain(buf, sem)
            issue(c0g + b + 3, nbuf, nsem)
            acc = process(buf, c0g + b, acc)
        return acc

    acc_init = tuple(jnp.zeros((16,), jnp.float32) for _ in range(16))
    nquads = lax.div(nc + 3, 4)
    acc = lax.fori_loop(0, nquads, quad_body, acc_init)

    @pl.when(nc > 0)
    def _():
        drain(buf_a, sem_a)
        drain(buf_b, sem_b)
        drain(buf_c, sem_c)

    inv0 = jnp.maximum((m_row - a_row).astype(jnp.float32), 1.0)
    inv1 = jnp.maximum((e_row - m_row).astype(jnp.float32), 1.0)
    for k in range(8):
        mean_v[pl.ds(k * 16, 16)] = acc[k] / inv0
        mean_v[pl.ds(D_FEAT_C + k * 16, 16)] = acc[8 + k] / inv1

    pltpu.sync_copy(mean_v, mean_hbm.at[pl.ds(wid * SEG_PER_W * D_FEAT_C, SEG_PER_W * D_FEAT_C)])


_sc_pool = functools.partial(
    pl.kernel,
    out_type=jax.ShapeDtypeStruct((N_SEG * D_FEAT_C,), jnp.float32),
    mesh=plsc.VectorSubcoreMesh(core_axis_name="c", subcore_axis_name="s"),
    compiler_params=pltpu.CompilerParams(needs_layout_passes=False),
    scratch_types=[
        pltpu.VMEM((PAD_N,), jnp.int32),
        pltpu.VMEM((CHUNK, D_FEAT_C), jnp.float32),
        pltpu.VMEM((CHUNK, D_FEAT_C), jnp.float32),
        pltpu.VMEM((CHUNK, D_FEAT_C), jnp.float32),
        pltpu.VMEM((CHUNK, D_FEAT_C), jnp.float32),
        pltpu.VMEM((SEG_PER_W * D_FEAT_C,), jnp.float32),
        pltpu.SemaphoreType.DMA,
        pltpu.SemaphoreType.DMA,
        pltpu.SemaphoreType.DMA,
        pltpu.SemaphoreType.DMA,
    ],
)(_sc_body)


def _mlp_body(u_ref, m_ref, w1_ref, b1_ref, w2_ref, b2_ref, o_ref):
    mean = m_ref[...].reshape(N_SEG, D_FEAT_C)
    n_global = u_ref.shape[1]
    w1u = w1_ref[0:n_global, :]
    w1x = w1_ref[n_global:, :]
    h = (
        jnp.dot(u_ref[...], w1u, preferred_element_type=jnp.float32)
        + jnp.dot(mean, w1x, preferred_element_type=jnp.float32)
        + b1_ref[...].reshape(1, -1)
    )
    h = jnp.maximum(h, 0.0)
    o_ref[...] = (
        jnp.dot(h, w2_ref[...], preferred_element_type=jnp.float32)
        + b2_ref[...].reshape(1, -1)
    )


def kernel(x, edge_index, edge_attr, u, batch, W1, b1, W2, b2):
    del edge_index, edge_attr
    mean_flat = _sc_pool(x, batch)
    out = pl.pallas_call(
        _mlp_body,
        out_shape=jax.ShapeDtypeStruct((u.shape[0], W2.shape[1]), jnp.float32),
    )(u, mean_flat, W1, b1, W2, b2)
    return out


# speedup vs baseline: 1.0311x; 1.0311x over previous
import functools

import jax
import jax.numpy as jnp
from jax import lax
from jax.experimental import pallas as pl
from jax.experimental.pallas import tpu as pltpu
from jax.experimental.pallas import tpu_sc as plsc

N_NODES_C = 10000
D_FEAT_C = 128
N_SEG = 64
SEG_PER_W = 2
CHUNK = 80
PAD_N = 10240


def _sc_body(x_hbm, batch_hbm, mean_hbm, batch_v, buf_a, buf_b, buf_c, buf_d,
             mean_v, sem_a, sem_b, sem_c, sem_d):
    wid = lax.axis_index("s") * 2 + lax.axis_index("c")

    pltpu.sync_copy(batch_hbm, batch_v.at[pl.ds(0, N_NODES_C)])
    pad_vec = jnp.full((16,), 127, dtype=jnp.int32)
    for i in range((PAD_N - N_NODES_C) // 16):
        batch_v[pl.ds(N_NODES_C + i * 16, 16)] = pad_vec

    def count_lt(t):
        def step(_, lohi):
            lo, hi = lohi
            mid = lax.div(lo + hi, 2)
            last = batch_v[pl.ds(mid * 16, 16)][15]
            found = last >= t
            return jnp.where(found, lo, mid + 1), jnp.where(found, mid, hi)

        lo, _ = lax.fori_loop(0, 10, step, (jnp.int32(0), jnp.int32(PAD_N // 16)))
        v = batch_v[pl.ds(lo * 16, 16)]
        return lo * 16 + jnp.sum(jnp.where(v < t, 1, 0))

    t0 = (wid * 2).astype(jnp.int32)
    a_row = count_lt(t0)
    m_row = count_lt(t0 + 1)
    e_row = count_lt(t0 + 2)

    c_start = lax.div(a_row, CHUNK)
    c_end = lax.div(e_row + (CHUNK - 1), CHUNK)
    nc = c_end - c_start

    def issue(c, buf, sem):
        base = jnp.minimum(c * CHUNK, N_NODES_C - CHUNK)
        pltpu.async_copy(x_hbm.at[pl.ds(base, CHUNK)], buf, sem)

    def drain(buf, sem):
        pltpu.make_async_copy(x_hbm.at[pl.ds(0, CHUNK)], buf, sem).wait()

    def process(buf, c, acc):
        base = jnp.minimum(c * CHUNK, N_NODES_C - CHUNK)
        row_lo = jnp.maximum(a_row, c * CHUNK)
        row_hi = jnp.minimum(e_row, (c + 1) * CHUNK)
        mid = jnp.minimum(jnp.maximum(m_row, row_lo), row_hi)

        def row(i, a):
            return tuple(a[k] + buf[i, pl.ds(k * 16, 16)] for k in range(8))

        acc0 = lax.fori_loop(row_lo - base, mid - base, row, tuple(acc[:8]))
        acc1 = lax.fori_loop(mid - base, row_hi - base, row, tuple(acc[8:]))
        return acc0 + acc1

    slots = ((buf_a, sem_a), (buf_b, sem_b), (buf_c, sem_c), (buf_d, sem_d))

    @pl.when(nc > 0)
    def _():
        issue(c_start, buf_a, sem_a)
        issue(c_start + 1, buf_b, sem_b)
        issue(c_start + 2, buf_c, sem_c)

    def quad_body(g, acc):
        c0g = c_start + 4 * g
        for b in range(4):
            buf, sem = slots[b]
            nbuf, nsem = slots[(b + 3) % 4]
            drain(buf, sem)
            issue(c0g + b + 3, nbuf, nsem)
            acc = process(buf, c0g + b, acc)
        return acc

    acc_init = tuple(jnp.zeros((16,), jnp.float32) for _ in range(16))
    nquads = lax.div(nc + 3, 4)
    acc = lax.fori_loop(0, nquads, quad_body, acc_init)

    @pl.when(nc > 0)
    def _():
        drain(buf_a, sem_a)
        drain(buf_b, sem_b)
        drain(buf_c, sem_c)

    inv0 = jnp.maximum((m_row - a_row).astype(jnp.float32), 1.0)
    inv1 = jnp.maximum((e_row - m_row).astype(jnp.float32), 1.0)
    for k in range(8):
        mean_v[pl.ds(k * 16, 16)] = acc[k] / inv0
        mean_v[pl.ds(D_FEAT_C + k * 16, 16)] = acc[8 + k] / inv1

    pltpu.sync_copy(mean_v, mean_hbm.at[pl.ds(wid * SEG_PER_W * D_FEAT_C, SEG_PER_W * D_FEAT_C)])


_sc_pool = functools.partial(
    pl.kernel,
    out_type=jax.ShapeDtypeStruct((N_SEG * D_FEAT_C,), jnp.float32),
    mesh=plsc.VectorSubcoreMesh(core_axis_name="c", subcore_axis_name="s"),
    compiler_params=pltpu.CompilerParams(needs_layout_passes=False),
    scratch_types=[
        pltpu.VMEM((PAD_N,), jnp.int32),
        pltpu.VMEM((CHUNK, D_FEAT_C), jnp.float32),
        pltpu.VMEM((CHUNK, D_FEAT_C), jnp.float32),
        pltpu.VMEM((CHUNK, D_FEAT_C), jnp.float32),
        pltpu.VMEM((CHUNK, D_FEAT_C), jnp.float32),
        pltpu.VMEM((SEG_PER_W * D_FEAT_C,), jnp.float32),
        pltpu.SemaphoreType.DMA,
        pltpu.SemaphoreType.DMA,
        pltpu.SemaphoreType.DMA,
        pltpu.SemaphoreType.DMA,
    ],
)(_sc_body)


def _mlp_body(u_ref, m_ref, w1_ref, b1_ref, w2_ref, b2_ref, o_ref):
    mean = m_ref[...].reshape(N_SEG, D_FEAT_C)
    n_global = u_ref.shape[1]
    w1u = w1_ref[0:n_global, :]
    w1x = w1_ref[n_global:, :]
    h = (
        jnp.dot(u_ref[...], w1u, preferred_element_type=jnp.float32)
        + jnp.dot(mean, w1x, preferred_element_type=jnp.float32)
        + b1_ref[...].reshape(1, -1)
    )
    h = jnp.maximum(h, 0.0)
    o_ref[...] = (
        jnp.dot(h, w2_ref[...], preferred_element_type=jnp.float32)
        + b2_ref[...].reshape(1, -1)
    )


def kernel(x, edge_index, edge_attr, u, batch, W1, b1, W2, b2):
    del edge_index, edge_attr
    mean_flat = _sc_pool(x, batch)
    out = pl.pallas_call(
        _mlp_body,
        out_shape=jax.ShapeDtypeStruct((u.shape[0], W2.shape[1]), jnp.float32),
    )(u, mean_flat, W1, b1, W2, b2)
    return out


# speedup vs baseline: 1.0968x; 1.0637x over previous
import functools

import jax
import jax.numpy as jnp
from jax import lax
from jax.experimental import pallas as pl
from jax.experimental.pallas import tpu as pltpu
from jax.experimental.pallas import tpu_sc as plsc

N_NODES_C = 10000
D_FEAT_C = 128
N_SEG = 64
SEG_PER_W = 2
CHUNK = 40
PAD_N = 10240


def _sc_body(x_hbm, batch_hbm, mean_hbm, batch_v, batch_sh, buf_a, buf_b, buf_c,
             buf_d, mean_v, sem_a, sem_b, sem_c, sem_d):
    sid = lax.axis_index("s")
    wid = sid * 2 + lax.axis_index("c")

    pad_vec = jnp.full((16,), 127, dtype=jnp.int32)

    @pl.when(sid == 0)
    def _():
        pltpu.sync_copy(batch_hbm, batch_sh)

    for i in range((PAD_N - N_NODES_C) // 16):
        batch_v[pl.ds(N_NODES_C + i * 16, 16)] = pad_vec
    plsc.subcore_barrier()
    pltpu.sync_copy(batch_sh, batch_v.at[pl.ds(0, N_NODES_C)])

    def count_lt(t):
        def step(_, lohi):
            lo, hi = lohi
            mid = lax.div(lo + hi, 2)
            last = batch_v[pl.ds(mid * 16, 16)][15]
            found = last >= t
            return jnp.where(found, lo, mid + 1), jnp.where(found, mid, hi)

        lo, _ = lax.fori_loop(0, 10, step, (jnp.int32(0), jnp.int32(PAD_N // 16)))
        v = batch_v[pl.ds(lo * 16, 16)]
        return lo * 16 + jnp.sum(jnp.where(v < t, 1, 0))

    t0 = (wid * 2).astype(jnp.int32)
    a_row = count_lt(t0)
    m_row = count_lt(t0 + 1)
    e_row = count_lt(t0 + 2)

    c_start = lax.div(a_row, CHUNK)
    c_end = lax.div(e_row + (CHUNK - 1), CHUNK)
    nc = c_end - c_start

    def issue(c, buf, sem):
        base = jnp.minimum(c * CHUNK, N_NODES_C - CHUNK)
        pltpu.async_copy(x_hbm.at[pl.ds(base, CHUNK)], buf, sem)

    def drain(buf, sem):
        pltpu.make_async_copy(x_hbm.at[pl.ds(0, CHUNK)], buf, sem).wait()

    def process(buf, c, acc):
        base = jnp.minimum(c * CHUNK, N_NODES_C - CHUNK)
        row_lo = jnp.maximum(a_row, c * CHUNK)
        row_hi = jnp.minimum(e_row, (c + 1) * CHUNK)
        mid = jnp.minimum(jnp.maximum(m_row, row_lo), row_hi)

        def row(i, a):
            return tuple(a[k] + buf[i, pl.ds(k * 16, 16)] for k in range(8))

        acc0 = lax.fori_loop(row_lo - base, mid - base, row, tuple(acc[:8]))
        acc1 = lax.fori_loop(mid - base, row_hi - base, row, tuple(acc[8:]))
        return acc0 + acc1

    slots = ((buf_a, sem_a), (buf_b, sem_b), (buf_c, sem_c), (buf_d, sem_d))

    @pl.when(nc > 0)
    def _():
        issue(c_start, buf_a, sem_a)
        issue(c_start + 1, buf_b, sem_b)
        issue(c_start + 2, buf_c, sem_c)

    def quad_body(g, acc):
        c0g = c_start + 4 * g
        for b in range(4):
            buf, sem = slots[b]
            nbuf, nsem = slots[(b + 3) % 4]
            drain(buf, sem)
            issue(c0g + b + 3, nbuf, nsem)
            acc = process(buf, c0g + b, acc)
        return acc

    acc_init = tuple(jnp.zeros((16,), jnp.float32) for _ in range(16))
    nquads = lax.div(nc + 3, 4)
    acc = lax.fori_loop(0, nquads, quad_body, acc_init)

    @pl.when(nc > 0)
    def _():
        drain(buf_a, sem_a)
        drain(buf_b, sem_b)
        drain(buf_c, sem_c)

    inv0 = jnp.maximum((m_row - a_row).astype(jnp.float32), 1.0)
    inv1 = jnp.maximum((e_row - m_row).astype(jnp.float32), 1.0)
    for k in range(8):
        mean_v[pl.ds(k * 16, 16)] = acc[k] / inv0
        mean_v[pl.ds(D_FEAT_C + k * 16, 16)] = acc[8 + k] / inv1

    pltpu.sync_copy(mean_v, mean_hbm.at[pl.ds(wid * SEG_PER_W * D_FEAT_C, SEG_PER_W * D_FEAT_C)])


_sc_pool = functools.partial(
    pl.kernel,
    out_type=jax.ShapeDtypeStruct((N_SEG * D_FEAT_C,), jnp.float32),
    mesh=plsc.VectorSubcoreMesh(core_axis_name="c", subcore_axis_name="s"),
    compiler_params=pltpu.CompilerParams(needs_layout_passes=False),
    scratch_types=[
        pltpu.VMEM((PAD_N,), jnp.int32),
        pltpu.VMEM_SHARED((N_NODES_C,), jnp.int32),
        pltpu.VMEM((CHUNK, D_FEAT_C), jnp.float32),
        pltpu.VMEM((CHUNK, D_FEAT_C), jnp.float32),
        pltpu.VMEM((CHUNK, D_FEAT_C), jnp.float32),
        pltpu.VMEM((CHUNK, D_FEAT_C), jnp.float32),
        pltpu.VMEM((SEG_PER_W * D_FEAT_C,), jnp.float32),
        pltpu.SemaphoreType.DMA,
        pltpu.SemaphoreType.DMA,
        pltpu.SemaphoreType.DMA,
        pltpu.SemaphoreType.DMA,
    ],
)(_sc_body)


def _mlp_body(u_ref, m_ref, w1_ref, b1_ref, w2_ref, b2_ref, o_ref):
    mean = m_ref[...].reshape(N_SEG, D_FEAT_C)
    n_global = u_ref.shape[1]
    w1u = w1_ref[0:n_global, :]
    w1x = w1_ref[n_global:, :]
    h = (
        jnp.dot(u_ref[...], w1u, preferred_element_type=jnp.float32)
        + jnp.dot(mean, w1x, preferred_element_type=jnp.float32)
        + b1_ref[...].reshape(1, -1)
    )
    h = jnp.maximum(h, 0.0)
    o_ref[...] = (
        jnp.dot(h, w2_ref[...], preferred_element_type=jnp.float32)
        + b2_ref[...].reshape(1, -1)
    )


def kernel(x, edge_index, edge_attr, u, batch, W1, b1, W2, b2):
    del edge_index, edge_attr
    mean_flat = _sc_pool(x, batch)
    out = pl.pallas_call(
        _mlp_body,
        out_shape=jax.ShapeDtypeStruct((u.shape[0], W2.shape[1]), jnp.float32),
    )(u, mean_flat, W1, b1, W2, b2)
    return out
